# baseline (speedup 1.0000x reference)
import functools

import jax
import jax.numpy as jnp
from jax import lax
from jax.experimental import pallas as pl
from jax.experimental.pallas import tpu as pltpu
from jax.experimental.pallas import tpu_sc as plsc

B, D, L, V, E = 16384, 13, 20, 1000000, 64

NC, NS, LANES = 2, 16, 16
NW = NC * NS
RPT = B // NW
IPT = RPT * L
CB = 16
CROWS = CB * L
NCH = RPT // CB


def _sc_pool_body(cat_hbm, table_hbm, out_hbm, idx_v, rows_v, out_v, sem):
    wid = lax.axis_index("s") * NC + lax.axis_index("c")
    pltpu.sync_copy(cat_hbm.at[pl.ds(wid * IPT, IPT)], idx_v)

    def chunk_body(c, _):
        pltpu.async_copy(
            table_hbm.at[idx_v.at[pl.ds(c * CROWS, CROWS)]], rows_v, sem
        ).wait()

        def row_body(b, _):
            accs = [rows_v[b * L, pl.ds(j * LANES, LANES)] for j in range(E // LANES)]
            for l in range(1, L):
                for j in range(E // LANES):
                    accs[j] = accs[j] + rows_v[b * L + l, pl.ds(j * LANES, LANES)]
            for j in range(E // LANES):
                out_v[c * CB + b, pl.ds(j * LANES, LANES)] = accs[j] * (1.0 / L)
            return 0

        lax.fori_loop(0, CB, row_body, 0)
        return 0

    lax.fori_loop(0, NCH, chunk_body, 0)
    pltpu.sync_copy(out_v, out_hbm.at[pl.ds(wid * RPT, RPT)])


def _make_sc_pool():
    mesh = plsc.VectorSubcoreMesh(core_axis_name="c", subcore_axis_name="s")
    return functools.partial(
        pl.kernel,
        mesh=mesh,
        out_type=jax.ShapeDtypeStruct((B, E), jnp.float32),
        scratch_types=[
            pltpu.VMEM((IPT,), jnp.int32),
            pltpu.VMEM((CROWS, E), jnp.float32),
            pltpu.VMEM((RPT, E), jnp.float32),
            pltpu.SemaphoreType.DMA,
        ],
        compiler_params=pltpu.CompilerParams(use_tc_tiling_on_sc=False),
    )(_sc_pool_body)


_sc_pool = _make_sc_pool()


BLK = 1024
NBLK = B // BLK


def _bottom_body(x_ref, w0_ref, b0_ref, w1_ref, b1_ref, w2_ref, b2_ref, out_ref):
    h = jnp.dot(x_ref[...], w0_ref[...], preferred_element_type=jnp.float32)
    h = jnp.maximum(h + b0_ref[...], 0.0)
    h = jnp.dot(h, w1_ref[...], preferred_element_type=jnp.float32)
    h = jnp.maximum(h + b1_ref[...], 0.0)
    h = jnp.dot(h, w2_ref[...], preferred_element_type=jnp.float32)
    out_ref[...] = jnp.maximum(h + b2_ref[...], 0.0)


def _top_body(de_ref, se_ref, wt0a_ref, wt0b_ref, bt0_ref, wt1_ref, bt1_ref,
              wt2_ref, bt2_ref, out_ref):
    de = de_ref[...]
    inter = jnp.sum(de * se_ref[...], axis=1, keepdims=True)
    t = jnp.dot(de, wt0a_ref[...], preferred_element_type=jnp.float32)
    t = jnp.maximum(t + inter * wt0b_ref[...] + bt0_ref[...], 0.0)
    t = jnp.dot(t, wt1_ref[...], preferred_element_type=jnp.float32)
    t = jnp.maximum(t + bt1_ref[...], 0.0)
    t = jnp.dot(t, wt2_ref[...], preferred_element_type=jnp.float32)
    out_ref[...] = t + bt2_ref[...]


def _full_spec(shape):
    return pl.BlockSpec(shape, lambda i: (0,) * len(shape))


def _bottom_mlp(x, w0, b0, w1, b1, w2, b2):
    return pl.pallas_call(
        _bottom_body,
        grid=(NBLK,),
        in_specs=[
            pl.BlockSpec((BLK, D), lambda i: (i, 0)),
            _full_spec((D, 512)), _full_spec((1, 512)),
            _full_spec((512, 256)), _full_spec((1, 256)),
            _full_spec((256, E)), _full_spec((1, E)),
        ],
        out_specs=pl.BlockSpec((BLK, E), lambda i: (i, 0)),
        out_shape=jax.ShapeDtypeStruct((B, E), jnp.float32),
    )(x, w0, b0, w1, b1, w2, b2)


def _top_mlp(dense_emb, sparse_emb, wt0a, wt0b, bt0, wt1, bt1, wt2, bt2):
    return pl.pallas_call(
        _top_body,
        grid=(NBLK,),
        in_specs=[
            pl.BlockSpec((BLK, E), lambda i: (i, 0)),
            pl.BlockSpec((BLK, E), lambda i: (i, 0)),
            _full_spec((E, 512)), _full_spec((1, 512)), _full_spec((1, 512)),
            _full_spec((512, 256)), _full_spec((1, 256)),
            _full_spec((256, 1)), _full_spec((1, 1)),
        ],
        out_specs=pl.BlockSpec((BLK, 1), lambda i: (i, 0)),
        out_shape=jax.ShapeDtypeStruct((B, 1), jnp.float32),
    )(dense_emb, sparse_emb, wt0a, wt0b, bt0, wt1, bt1, wt2, bt2)


def kernel(dense_features, category_ids, W0, b0, W1, b1, W2, b2, emb_table,
           Wt0, bt0, Wt1, bt1, Wt2, bt2):
    cat_flat = category_ids.reshape(-1)
    sparse_emb = _sc_pool(cat_flat, emb_table)
    dense_emb = _bottom_mlp(
        dense_features, W0, b0.reshape(1, -1), W1, b1.reshape(1, -1),
        W2, b2.reshape(1, -1))
    logits = _top_mlp(
        dense_emb, sparse_emb, Wt0[:E], Wt0[E:E + 1], bt0.reshape(1, -1),
        Wt1, bt1.reshape(1, -1), Wt2, bt2.reshape(1, -1))
    return logits.reshape(-1)


# speedup vs baseline: 1.2348x; 1.2348x over previous
import functools

import jax
import jax.numpy as jnp
from jax import lax
from jax.experimental import pallas as pl
from jax.experimental.pallas import tpu as pltpu
from jax.experimental.pallas import tpu_sc as plsc

B, D, L, V, E = 16384, 13, 20, 1000000, 64

NC, NS, LANES = 2, 16, 16
NW = NC * NS
RPT = B // NW
IPT = RPT * L
CB = 16
CROWS = CB * L
NCH = RPT // CB


CIN = 2048
CGRID = (V + CIN - 1) // CIN


def _conv_body(tt_ref, out_ref):
    t = tt_ref[...].T
    out_ref[...] = jnp.concatenate([t[:CIN // 2], t[CIN // 2:]], axis=1)


def _convert_table(table_t):
    return pl.pallas_call(
        _conv_body,
        grid=(CGRID,),
        in_specs=[pl.BlockSpec((E, CIN), lambda j: (0, j))],
        out_specs=pl.BlockSpec((CIN // 2, 2 * E), lambda j: (j, 0)),
        out_shape=jax.ShapeDtypeStruct((CGRID * CIN // 2, 2 * E), jnp.float32),
    )(table_t)


def _sc_pool_body(cat_hbm, table_hbm, out_hbm, idx_v, m_v, rows_v, out_v, sem):
    wid = lax.axis_index("s") * NC + lax.axis_index("c")
    pltpu.sync_copy(cat_hbm.at[pl.ds(wid * IPT, IPT)], idx_v)

    def prep(s, _):
        ids = idx_v[pl.ds(s * LANES, LANES)]
        m_v[pl.ds(s * LANES, LANES)] = (
            jnp.left_shift(jnp.right_shift(ids, 11), 11)
            | jnp.left_shift(jnp.bitwise_and(ids, 1023), 1)
            | jnp.bitwise_and(jnp.right_shift(ids, 10), 1)
        )
        return 0

    lax.fori_loop(0, IPT // LANES, prep, 0)

    def chunk_body(c, _):
        pltpu.async_copy(
            table_hbm.at[m_v.at[pl.ds(c * CROWS, CROWS)]], rows_v, sem
        ).wait()

        def row_body(b, _):
            accs = [rows_v[b * L, pl.ds(j * LANES, LANES)] for j in range(E // LANES)]
            for l in range(1, L):
                for j in range(E // LANES):
                    accs[j] = accs[j] + rows_v[b * L + l, pl.ds(j * LANES, LANES)]
            for j in range(E // LANES):
                out_v[c * CB + b, pl.ds(j * LANES, LANES)] = accs[j] * (1.0 / L)
            return 0

        lax.fori_loop(0, CB, row_body, 0)
        return 0

    lax.fori_loop(0, NCH, chunk_body, 0)
    pltpu.sync_copy(out_v, out_hbm.at[pl.ds(wid * RPT, RPT)])


def _make_sc_pool():
    mesh = plsc.VectorSubcoreMesh(core_axis_name="c", subcore_axis_name="s")
    return functools.partial(
        pl.kernel,
        mesh=mesh,
        out_type=jax.ShapeDtypeStruct((B, E), jnp.float32),
        scratch_types=[
            pltpu.VMEM((IPT,), jnp.int32),
            pltpu.VMEM((IPT,), jnp.int32),
            pltpu.VMEM((CROWS, E), jnp.float32),
            pltpu.VMEM((RPT, E), jnp.float32),
            pltpu.SemaphoreType.DMA,
        ],
        compiler_params=pltpu.CompilerParams(use_tc_tiling_on_sc=False),
    )(_sc_pool_body)


_sc_pool = _make_sc_pool()


BLK = 1024
NBLK = B // BLK


def _bottom_body(x_ref, w0_ref, b0_ref, w1_ref, b1_ref, w2_ref, b2_ref, out_ref):
    h = jnp.dot(x_ref[...], w0_ref[...], preferred_element_type=jnp.float32)
    h = jnp.maximum(h + b0_ref[...], 0.0)
    h = jnp.dot(h, w1_ref[...], preferred_element_type=jnp.float32)
    h = jnp.maximum(h + b1_ref[...], 0.0)
    h = jnp.dot(h, w2_ref[...], preferred_element_type=jnp.float32)
    out_ref[...] = jnp.maximum(h + b2_ref[...], 0.0)


def _top_body(de_ref, se_ref, wt0a_ref, wt0b_ref, bt0_ref, wt1_ref, bt1_ref,
              wt2_ref, bt2_ref, out_ref):
    de = de_ref[...]
    inter = jnp.sum(de * se_ref[...], axis=1, keepdims=True)
    t = jnp.dot(de, wt0a_ref[...], preferred_element_type=jnp.float32)
    t = jnp.maximum(t + inter * wt0b_ref[...] + bt0_ref[...], 0.0)
    t = jnp.dot(t, wt1_ref[...], preferred_element_type=jnp.float32)
    t = jnp.maximum(t + bt1_ref[...], 0.0)
    t = jnp.dot(t, wt2_ref[...], preferred_element_type=jnp.float32)
    out_ref[...] = t + bt2_ref[...]


def _full_spec(shape):
    return pl.BlockSpec(shape, lambda i: (0,) * len(shape))


def _bottom_mlp(x, w0, b0, w1, b1, w2, b2):
    return pl.pallas_call(
        _bottom_body,
        grid=(NBLK,),
        in_specs=[
            pl.BlockSpec((BLK, D), lambda i: (i, 0)),
            _full_spec((D, 512)), _full_spec((1, 512)),
            _full_spec((512, 256)), _full_spec((1, 256)),
            _full_spec((256, E)), _full_spec((1, E)),
        ],
        out_specs=pl.BlockSpec((BLK, E), lambda i: (i, 0)),
        out_shape=jax.ShapeDtypeStruct((B, E), jnp.float32),
    )(x, w0, b0, w1, b1, w2, b2)


def _top_mlp(dense_emb, sparse_emb, wt0a, wt0b, bt0, wt1, bt1, wt2, bt2):
    return pl.pallas_call(
        _top_body,
        grid=(NBLK,),
        in_specs=[
            pl.BlockSpec((BLK, E), lambda i: (i, 0)),
            pl.BlockSpec((BLK, E), lambda i: (i, 0)),
            _full_spec((E, 512)), _full_spec((1, 512)), _full_spec((1, 512)),
            _full_spec((512, 256)), _full_spec((1, 256)),
            _full_spec((256, 1)), _full_spec((1, 1)),
        ],
        out_specs=pl.BlockSpec((BLK, 1), lambda i: (i, 0)),
        out_shape=jax.ShapeDtypeStruct((B, 1), jnp.float32),
    )(dense_emb, sparse_emb, wt0a, wt0b, bt0, wt1, bt1, wt2, bt2)


def kernel(dense_features, category_ids, W0, b0, W1, b1, W2, b2, emb_table,
           Wt0, bt0, Wt1, bt1, Wt2, bt2):
    cat_flat = category_ids.reshape(-1)
    t128 = _convert_table(jnp.transpose(emb_table))
    t_lin = t128.reshape(CGRID * CIN, E)
    sparse_emb = _sc_pool(cat_flat, t_lin)
    dense_emb = _bottom_mlp(
        dense_features, W0, b0.reshape(1, -1), W1, b1.reshape(1, -1),
        W2, b2.reshape(1, -1))
    logits = _top_mlp(
        dense_emb, sparse_emb, Wt0[:E], Wt0[E:E + 1], bt0.reshape(1, -1),
        Wt1, bt1.reshape(1, -1), Wt2, bt2.reshape(1, -1))
    return logits.reshape(-1)


# speedup vs baseline: 1.5567x; 1.2607x over previous
import functools

import jax
import jax.numpy as jnp
from jax import lax
from jax.experimental import pallas as pl
from jax.experimental.pallas import tpu as pltpu
from jax.experimental.pallas import tpu_sc as plsc

B, D, L, V, E = 16384, 13, 20, 1000000, 64

NC, NS, LANES = 2, 16, 16
NW = NC * NS
RPT = B // NW
IPT = RPT * L
CB = 16
CROWS = CB * L
NCH = RPT // CB


CIN = 4096
CGRID = (V + CIN - 1) // CIN
CSH = 12


def _conv_body(tt_ref, out_ref):
    eye = jnp.eye(E, dtype=jnp.float32)
    t = lax.dot_general(tt_ref[...], eye, (((0,), (0,)), ((), ())),
                        preferred_element_type=jnp.float32)
    out_ref[...] = jnp.concatenate([t[:CIN // 2], t[CIN // 2:]], axis=1)


def _convert_table(table_t):
    return pl.pallas_call(
        _conv_body,
        grid=(CGRID,),
        in_specs=[pl.BlockSpec((E, CIN), lambda j: (0, j))],
        out_specs=pl.BlockSpec((CIN // 2, 2 * E), lambda j: (j, 0)),
        out_shape=jax.ShapeDtypeStruct((CGRID * CIN // 2, 2 * E), jnp.float32),
    )(table_t)


def _sc_pool_body(cat_hbm, table_hbm, out_hbm, idx_v, m_v, rows_v, out_v, sem):
    wid = lax.axis_index("s") * NC + lax.axis_index("c")
    pltpu.sync_copy(cat_hbm.at[pl.ds(wid * IPT, IPT)], idx_v)

    def prep(s, _):
        ids = idx_v[pl.ds(s * LANES, LANES)]
        m_v[pl.ds(s * LANES, LANES)] = (
            jnp.left_shift(jnp.right_shift(ids, CSH), CSH)
            | jnp.left_shift(jnp.bitwise_and(ids, CIN // 2 - 1), 1)
            | jnp.bitwise_and(jnp.right_shift(ids, CSH - 1), 1)
        )
        return 0

    lax.fori_loop(0, IPT // LANES, prep, 0)

    def chunk_body(c, _):
        pltpu.async_copy(
            table_hbm.at[m_v.at[pl.ds(c * CROWS, CROWS)]], rows_v, sem
        ).wait()

        def row_body(b, _):
            accs = [rows_v[b * L, pl.ds(j * LANES, LANES)] for j in range(E // LANES)]
            for l in range(1, L):
                for j in range(E // LANES):
                    accs[j] = accs[j] + rows_v[b * L + l, pl.ds(j * LANES, LANES)]
            for j in range(E // LANES):
                out_v[c * CB + b, pl.ds(j * LANES, LANES)] = accs[j] * (1.0 / L)
            return 0

        lax.fori_loop(0, CB, row_body, 0)
        return 0

    lax.fori_loop(0, NCH, chunk_body, 0)
    pltpu.sync_copy(out_v, out_hbm.at[pl.ds(wid * RPT, RPT)])


def _make_sc_pool():
    mesh = plsc.VectorSubcoreMesh(core_axis_name="c", subcore_axis_name="s")
    return functools.partial(
        pl.kernel,
        mesh=mesh,
        out_type=jax.ShapeDtypeStruct((B, E), jnp.float32),
        scratch_types=[
            pltpu.VMEM((IPT,), jnp.int32),
            pltpu.VMEM((IPT,), jnp.int32),
            pltpu.VMEM((CROWS, E), jnp.float32),
            pltpu.VMEM((RPT, E), jnp.float32),
            pltpu.SemaphoreType.DMA,
        ],
        compiler_params=pltpu.CompilerParams(use_tc_tiling_on_sc=False),
    )(_sc_pool_body)


_sc_pool = _make_sc_pool()


BLK = 1024
NBLK = B // BLK


def _bottom_body(x_ref, w0_ref, b0_ref, w1_ref, b1_ref, w2_ref, b2_ref, out_ref):
    h = jnp.dot(x_ref[...], w0_ref[...], preferred_element_type=jnp.float32)
    h = jnp.maximum(h + b0_ref[...], 0.0)
    h = jnp.dot(h, w1_ref[...], preferred_element_type=jnp.float32)
    h = jnp.maximum(h + b1_ref[...], 0.0)
    h = jnp.dot(h, w2_ref[...], preferred_element_type=jnp.float32)
    out_ref[...] = jnp.maximum(h + b2_ref[...], 0.0)


def _top_body(de_ref, se_ref, wt0a_ref, wt0b_ref, bt0_ref, wt1_ref, bt1_ref,
              wt2_ref, bt2_ref, out_ref):
    de = de_ref[...]
    inter = jnp.sum(de * se_ref[...], axis=1, keepdims=True)
    t = jnp.dot(de, wt0a_ref[...], preferred_element_type=jnp.float32)
    t = jnp.maximum(t + inter * wt0b_ref[...] + bt0_ref[...], 0.0)
    t = jnp.dot(t, wt1_ref[...], preferred_element_type=jnp.float32)
    t = jnp.maximum(t + bt1_ref[...], 0.0)
    t = jnp.dot(t, wt2_ref[...], preferred_element_type=jnp.float32)
    out_ref[...] = t + bt2_ref[...]


def _full_spec(shape):
    return pl.BlockSpec(shape, lambda i: (0,) * len(shape))


def _bottom_mlp(x, w0, b0, w1, b1, w2, b2):
    return pl.pallas_call(
        _bottom_body,
        grid=(NBLK,),
        in_specs=[
            pl.BlockSpec((BLK, D), lambda i: (i, 0)),
            _full_spec((D, 512)), _full_spec((1, 512)),
            _full_spec((512, 256)), _full_spec((1, 256)),
            _full_spec((256, E)), _full_spec((1, E)),
        ],
        out_specs=pl.BlockSpec((BLK, E), lambda i: (i, 0)),
        out_shape=jax.ShapeDtypeStruct((B, E), jnp.float32),
    )(x, w0, b0, w1, b1, w2, b2)


def _top_mlp(dense_emb, sparse_emb, wt0a, wt0b, bt0, wt1, bt1, wt2, bt2):
    return pl.pallas_call(
        _top_body,
        grid=(NBLK,),
        in_specs=[
            pl.BlockSpec((BLK, E), lambda i: (i, 0)),
            pl.BlockSpec((BLK, E), lambda i: (i, 0)),
            _full_spec((E, 512)), _full_spec((1, 512)), _full_spec((1, 512)),
            _full_spec((512, 256)), _full_spec((1, 256)),
            _full_spec((256, 1)), _full_spec((1, 1)),
        ],
        out_specs=pl.BlockSpec((BLK, 1), lambda i: (i, 0)),
        out_shape=jax.ShapeDtypeStruct((B, 1), jnp.float32),
    )(dense_emb, sparse_emb, wt0a, wt0b, bt0, wt1, bt1, wt2, bt2)


def kernel(dense_features, category_ids, W0, b0, W1, b1, W2, b2, emb_table,
           Wt0, bt0, Wt1, bt1, Wt2, bt2):
    cat_flat = category_ids.reshape(-1)
    t128 = _convert_table(jnp.transpose(emb_table))
    t_lin = t128.reshape(CGRID * CIN, E)
    sparse_emb = _sc_pool(cat_flat, t_lin)
    dense_emb = _bottom_mlp(
        dense_features, W0, b0.reshape(1, -1), W1, b1.reshape(1, -1),
        W2, b2.reshape(1, -1))
    logits = _top_mlp(
        dense_emb, sparse_emb, Wt0[:E], Wt0[E:E + 1], bt0.reshape(1, -1),
        Wt1, bt1.reshape(1, -1), Wt2, bt2.reshape(1, -1))
    return logits.reshape(-1)
